# baseline (speedup 1.0000x reference)
import jax
import jax.numpy as jnp
from jax import lax
from jax.experimental import pallas as pl
from jax.experimental.pallas import tpu as pltpu
from jax.experimental.pallas import tpu_sc as plsc

BS = 128
BIN_SIZE = 2048
DIM = 256
NC = 2
NS = 16
SPW = BS // (NC * NS)
REP = 256
NCH = BIN_SIZE // REP
NLANE = DIM // 16


def _sc_kernel(idx_hbm, ce_hbm, out_hbm, idx_v, rows_v, rep, sem, gsem):
    wid = lax.axis_index("s") * NC + lax.axis_index("c")
    cb = (wid // 4) * 16
    pltpu.sync_copy(idx_hbm.at[pl.ds(cb, 16)], idx_v)
    pltpu.async_copy(ce_hbm.at[idx_v], rows_v, gsem).wait()
    base_l = (wid % 4) * 4

    def per_sample(k, carry):
        l = base_l + k
        i = cb + l
        def fill_row(r, c2):
            for j in range(NLANE):
                rep[r, pl.ds(16 * j, 16)] = rows_v[l, pl.ds(16 * j, 16)]
            return c2

        lax.fori_loop(0, REP, fill_row, 0)
        for c in range(NCH):
            pltpu.make_async_copy(
                rep, out_hbm.at[i, pl.ds(c * REP, REP), :], sem
            ).start()
        for c in range(NCH):
            pltpu.make_async_copy(
                rep, out_hbm.at[i, pl.ds(c * REP, REP), :], sem
            ).wait()
        pltpu.sync_copy(
            rows_v.at[pl.ds(l, 1), :],
            out_hbm.at[i, pl.ds(BIN_SIZE, 1), :],
        )
        return carry

    lax.fori_loop(0, SPW, per_sample, 0)


def kernel(tensor, chrom, ce):
    del tensor
    idx = chrom.astype(jnp.int32) - 1
    mesh = plsc.VectorSubcoreMesh(core_axis_name="c", subcore_axis_name="s")
    return pl.kernel(
        _sc_kernel,
        out_type=jax.ShapeDtypeStruct((BS, BIN_SIZE + 1, DIM), jnp.float32),
        mesh=mesh,
        scratch_types=[
            pltpu.VMEM((16,), jnp.int32),
            pltpu.VMEM((16, DIM), jnp.float32),
            pltpu.VMEM((REP, DIM), jnp.float32),
            pltpu.SemaphoreType.DMA,
            pltpu.SemaphoreType.DMA,
        ],
        compiler_params=pltpu.CompilerParams(use_tc_tiling_on_sc=True),
    )(idx, ce)


# speedup vs baseline: 1.1589x; 1.1589x over previous
import jax
import jax.numpy as jnp
from jax import lax
from jax.experimental import pallas as pl
from jax.experimental.pallas import tpu as pltpu
from jax.experimental.pallas import tpu_sc as plsc

BS = 128
BIN_SIZE = 2048
DIM = 256
NBUF = 6
NC = 2
NS = 16
SAMPLES_PER_WORKER = BS // (NC * NS)


def _sc_tail_kernel(idx_hbm, ce_hbm, out_hbm, idx_v, rows_v, sem):
    wid = lax.axis_index("s") * NC + lax.axis_index("c")
    cb = (wid // 4) * 16
    pltpu.sync_copy(idx_hbm.at[pl.ds(cb, 16)], idx_v)
    pltpu.async_copy(ce_hbm.at[idx_v], rows_v, sem).wait()
    base_l = (wid % 4) * 4
    for k in range(SAMPLES_PER_WORKER):
        l = base_l + k
        i = cb + l
        pltpu.sync_copy(
            rows_v.at[pl.ds(l, 1), :],
            out_hbm.at[i, pl.ds(BIN_SIZE, 1), :],
        )


def _sc_tails(idx, ce):
    mesh = plsc.VectorSubcoreMesh(core_axis_name="c", subcore_axis_name="s")
    return pl.kernel(
        _sc_tail_kernel,
        out_type=jax.ShapeDtypeStruct((BS, BIN_SIZE + 1, DIM), jnp.float32),
        mesh=mesh,
        scratch_types=[
            pltpu.VMEM((16,), jnp.int32),
            pltpu.VMEM((16, DIM), jnp.float32),
            pltpu.SemaphoreType.DMA,
        ],
        compiler_params=pltpu.CompilerParams(use_tc_tiling_on_sc=True),
    )(idx, ce)


def _tc_body(idx_ref, ce_ref, tails_ref, out_ref, bufs, sems):
    del tails_ref

    def big_copy(slot, i):
        return pltpu.make_async_copy(
            bufs.at[slot], out_ref.at[i, pl.ds(0, BIN_SIZE), :], sems.at[slot]
        )

    def step(i, carry):
        slot = jax.lax.rem(i, NBUF)

        @pl.when(i >= NBUF)
        def _():
            big_copy(slot, i - NBUF).wait()

        row = idx_ref[i]
        bufs[pl.ds(slot, 1), :, :] = jnp.broadcast_to(
            ce_ref[row, :].reshape(1, 1, DIM), (1, BIN_SIZE, DIM)
        )
        big_copy(slot, i).start()
        return carry

    jax.lax.fori_loop(0, BS, step, 0)

    def drain(j, carry):
        i = BS - NBUF + j
        big_copy(jax.lax.rem(i, NBUF), i).wait()
        return carry

    jax.lax.fori_loop(0, NBUF, drain, 0)


def kernel(tensor, chrom, ce):
    del tensor
    idx = chrom.astype(jnp.int32) - 1
    tails = _sc_tails(idx, ce)
    grid_spec = pltpu.PrefetchScalarGridSpec(
        num_scalar_prefetch=1,
        grid=(1,),
        in_specs=[
            pl.BlockSpec((24, DIM), lambda i, idx_ref: (0, 0)),
            pl.BlockSpec(memory_space=pl.ANY),
        ],
        out_specs=pl.BlockSpec(memory_space=pl.ANY),
        scratch_shapes=[
            pltpu.VMEM((NBUF, BIN_SIZE, DIM), jnp.float32),
            pltpu.SemaphoreType.DMA((NBUF,)),
        ],
    )
    return pl.pallas_call(
        _tc_body,
        grid_spec=grid_spec,
        out_shape=jax.ShapeDtypeStruct((BS, BIN_SIZE + 1, DIM), jnp.float32),
        input_output_aliases={2: 0},
    )(idx, ce, tails)
